# baseline (speedup 1.0000x reference)
import jax
import jax.numpy as jnp
from jax import lax
from jax.experimental import pallas as pl
from jax.experimental.pallas import tpu as pltpu
from jax.experimental.pallas import tpu_sc as plsc

_NC = 2
_NS = 16
_L = 16
_NW = _NC * _NS
_D = 1024
_SLICES = _D // _L
_ROWS = 32 * 1024
_CHUNK = 32
_NACC = 8

_TC_ROWS = 24576
_TC_BLOCK = 4096


def _tc_body(x_ref, o_ref):
    xb = x_ref[...]
    minval = jnp.min(xb, axis=1, keepdims=True)
    iota = lax.broadcasted_iota(jnp.int32, xb.shape, 1).astype(jnp.float32)
    idx = jnp.min(jnp.where(xb == minval, iota, float(_D)), axis=1)
    o_ref[...] = idx.astype(jnp.int32)


def _tc_argmin(xf, nrows):
    return pl.pallas_call(
        _tc_body,
        grid=(nrows // _TC_BLOCK,),
        compiler_params=pltpu.CompilerParams(skip_device_barrier=True),
        in_specs=[pl.BlockSpec((_TC_BLOCK, _D), lambda i: (i, 0))],
        out_specs=pl.BlockSpec((_TC_BLOCK,), lambda i: (i,)),
        out_shape=jax.ShapeDtypeStruct((nrows,), jnp.int32),
    )(xf)


def _merge(va, sa, vb, sb):
    take_b = (vb < va) | ((vb == va) & (sb < sa))
    return jnp.where(take_b, vb, va), jnp.where(take_b, sb, sa)


def _sc_body(rows_per_w, x_hbm, o_hbm, buf, out_v, sem0, sem1):
    nchunks = rows_per_w // _CHUNK
    wid = lax.axis_index("s") * _NC + lax.axis_index("c")
    base_row = _TC_ROWS + wid * rows_per_w
    sems = (sem0, sem1)

    lane = lax.iota(jnp.int32, _L)
    inf16 = jnp.full((_L,), jnp.inf, jnp.float32)
    zero16 = jnp.zeros((_L,), jnp.int32)

    def copy_chunk(c, par):
        return pltpu.make_async_copy(
            x_hbm.at[pl.ds(base_row + c * _CHUNK, _CHUNK), :],
            buf.at[pl.ds(par * _CHUNK, _CHUNK), :],
            sems[par],
        )

    copy_chunk(0, 0).start()
    copy_chunk(1, 1).start()

    def do_chunk(c, par):
        copy_chunk(c, par).wait()

        def row_body(r, res_vec):
            lr = par * _CHUNK + r

            vals = [inf16] * _NACC
            bests = [zero16] * _NACC
            for s in range(_SLICES):
                j = s % _NACC
                v = buf[lr, pl.ds(s * _L, _L)]
                pred = v < vals[j]
                vals[j] = jnp.where(pred, v, vals[j])
                bests[j] = jnp.where(pred, jnp.full((_L,), s, jnp.int32),
                                     bests[j])

            while len(vals) > 1:
                nv, nb = [], []
                for j in range(0, len(vals), 2):
                    mv, mb = _merge(vals[j], bests[j], vals[j + 1], bests[j + 1])
                    nv.append(mv)
                    nb.append(mb)
                vals, bests = nv, nb
            vm, sm = vals[0], bests[0]
            idx16 = sm * _L + lane
            m = lax.reduce_min(vm, (0,))
            idxc = jnp.where(vm == m, idx16, _D)
            best = lax.reduce_min(idxc, (0,))
            res_vec = jnp.where(lane == (r & (_L - 1)),
                                jnp.full((_L,), best, jnp.int32), res_vec)

            @pl.when((r & (_L - 1)) == _L - 1)
            def _():
                out_v[pl.ds(c * _CHUNK + (r & ~(_L - 1)), _L)] = res_vec

            return res_vec

        lax.fori_loop(0, _CHUNK, row_body, zero16)

        @pl.when(c + 2 < nchunks)
        def _():
            copy_chunk(c + 2, par).start()

    def pair_body(p, carry):
        do_chunk(2 * p, 0)
        do_chunk(2 * p + 1, 1)
        return carry

    lax.fori_loop(0, nchunks // 2, pair_body, 0)
    pltpu.sync_copy(out_v, o_hbm.at[pl.ds(wid * rows_per_w, rows_per_w)])


def _sc_argmin(xf, nrows):
    rows_per_w = nrows // _NW
    mesh = plsc.VectorSubcoreMesh(core_axis_name="c", subcore_axis_name="s")
    f = pl.kernel(
        lambda *args: _sc_body(rows_per_w, *args),
        out_type=jax.ShapeDtypeStruct((nrows,), jnp.int32),
        mesh=mesh,
        compiler_params=pltpu.CompilerParams(
            needs_layout_passes=False, use_tc_tiling_on_sc=True,
            skip_device_barrier=True),
        scratch_types=[
            pltpu.VMEM((2 * _CHUNK, _D), jnp.float32),
            pltpu.VMEM((rows_per_w,), jnp.int32),
            pltpu.SemaphoreType.DMA,
            pltpu.SemaphoreType.DMA,
        ],
    )
    return f(xf)


def kernel(x):
    b, d1, d2 = x.shape
    xf = x.reshape(b * d1, d2)
    out_sc = _sc_argmin(xf, _ROWS - _TC_ROWS)
    out_tc = _tc_argmin(xf, _TC_ROWS)
    return jnp.concatenate([out_tc, out_sc]).reshape(b, d1)


# speedup vs baseline: 1.0241x; 1.0241x over previous
import jax
import jax.numpy as jnp
from jax import lax
from jax.experimental import pallas as pl
from jax.experimental.pallas import tpu as pltpu
from jax.experimental.pallas import tpu_sc as plsc

_NC = 2
_NS = 16
_L = 16
_NW = _NC * _NS
_D = 1024
_SLICES = _D // _L
_ROWS = 32 * 1024
_CHUNK = 32
_NACC = 8

_TC_ROWS = 20480
_TC_BLOCK = 2048


def _tc_body(x_ref, o_ref):
    xb = x_ref[...]
    minval = jnp.min(xb, axis=1, keepdims=True)
    iota = lax.broadcasted_iota(jnp.int32, xb.shape, 1).astype(jnp.float32)
    idx = jnp.min(jnp.where(xb == minval, iota, float(_D)), axis=1)
    o_ref[...] = idx.astype(jnp.int32)


def _tc_argmin(xf, nrows):
    return pl.pallas_call(
        _tc_body,
        grid=(nrows // _TC_BLOCK,),
        compiler_params=pltpu.CompilerParams(skip_device_barrier=True),
        in_specs=[pl.BlockSpec((_TC_BLOCK, _D), lambda i: (i, 0))],
        out_specs=pl.BlockSpec((_TC_BLOCK,), lambda i: (i,)),
        out_shape=jax.ShapeDtypeStruct((nrows,), jnp.int32),
    )(xf)


def _merge(va, sa, vb, sb):
    take_b = (vb < va) | ((vb == va) & (sb < sa))
    return jnp.where(take_b, vb, va), jnp.where(take_b, sb, sa)


def _sc_body(rows_per_w, x_hbm, o_hbm, buf, out_v, sem0, sem1):
    nchunks = rows_per_w // _CHUNK
    wid = lax.axis_index("s") * _NC + lax.axis_index("c")
    base_row = _TC_ROWS + wid * rows_per_w
    sems = (sem0, sem1)

    lane = lax.iota(jnp.int32, _L)
    inf16 = jnp.full((_L,), jnp.inf, jnp.float32)
    zero16 = jnp.zeros((_L,), jnp.int32)

    def copy_chunk(c, par):
        return pltpu.make_async_copy(
            x_hbm.at[pl.ds(base_row + c * _CHUNK, _CHUNK), :],
            buf.at[pl.ds(par * _CHUNK, _CHUNK), :],
            sems[par],
        )

    copy_chunk(0, 0).start()
    copy_chunk(1, 1).start()

    def do_chunk(c, par):
        copy_chunk(c, par).wait()

        def row_body(r, res_vec):
            lr = par * _CHUNK + r

            vals = [inf16] * _NACC
            bests = [zero16] * _NACC
            for s in range(_SLICES):
                j = s % _NACC
                v = buf[lr, pl.ds(s * _L, _L)]
                pred = v < vals[j]
                vals[j] = jnp.where(pred, v, vals[j])
                bests[j] = jnp.where(pred, jnp.full((_L,), s, jnp.int32),
                                     bests[j])

            while len(vals) > 1:
                nv, nb = [], []
                for j in range(0, len(vals), 2):
                    mv, mb = _merge(vals[j], bests[j], vals[j + 1], bests[j + 1])
                    nv.append(mv)
                    nb.append(mb)
                vals, bests = nv, nb
            vm, sm = vals[0], bests[0]
            idx16 = sm * _L + lane
            m = lax.reduce_min(vm, (0,))
            idxc = jnp.where(vm == m, idx16, _D)
            best = lax.reduce_min(idxc, (0,))
            res_vec = jnp.where(lane == (r & (_L - 1)),
                                jnp.full((_L,), best, jnp.int32), res_vec)

            @pl.when((r & (_L - 1)) == _L - 1)
            def _():
                out_v[pl.ds(c * _CHUNK + (r & ~(_L - 1)), _L)] = res_vec

            return res_vec

        lax.fori_loop(0, _CHUNK, row_body, zero16)

        @pl.when(c + 2 < nchunks)
        def _():
            copy_chunk(c + 2, par).start()

    def pair_body(p, carry):
        do_chunk(2 * p, 0)
        do_chunk(2 * p + 1, 1)
        return carry

    lax.fori_loop(0, nchunks // 2, pair_body, 0)
    pltpu.sync_copy(out_v, o_hbm.at[pl.ds(wid * rows_per_w, rows_per_w)])


def _sc_argmin(xf, nrows):
    rows_per_w = nrows // _NW
    mesh = plsc.VectorSubcoreMesh(core_axis_name="c", subcore_axis_name="s")
    f = pl.kernel(
        lambda *args: _sc_body(rows_per_w, *args),
        out_type=jax.ShapeDtypeStruct((nrows,), jnp.int32),
        mesh=mesh,
        compiler_params=pltpu.CompilerParams(
            needs_layout_passes=False, use_tc_tiling_on_sc=True,
            skip_device_barrier=True),
        scratch_types=[
            pltpu.VMEM((2 * _CHUNK, _D), jnp.float32),
            pltpu.VMEM((rows_per_w,), jnp.int32),
            pltpu.SemaphoreType.DMA,
            pltpu.SemaphoreType.DMA,
        ],
    )
    return f(xf)


def kernel(x):
    b, d1, d2 = x.shape
    xf = x.reshape(b * d1, d2)
    out_sc = _sc_argmin(xf, _ROWS - _TC_ROWS)
    out_tc = _tc_argmin(xf, _TC_ROWS)
    return jnp.concatenate([out_tc, out_sc]).reshape(b, d1)
